# baseline (speedup 1.0000x reference)
import functools

import jax
import jax.numpy as jnp
from jax import lax
from jax.experimental import pallas as pl
from jax.experimental.pallas import tpu as pltpu
from jax.experimental.pallas import tpu_sc as plsc

_NC = 2
_NS = 16
_NW = _NC * _NS


def _as_wide_i16(table):
    n, d = table.shape
    return jax.lax.bitcast_convert_type(table, jnp.int16).reshape(n, 2 * d)


def kernel(users, pos_items, neg_items, user_table, item_table):
    B = users.shape[0]
    D = user_table.shape[1]
    W = 2 * D
    b_per_w = B // _NW

    u_idx = users.astype(jnp.int32)
    p_idx = pos_items.astype(jnp.int32)
    n_idx = neg_items.astype(jnp.int32)
    ut = _as_wide_i16(user_table)
    it = _as_wide_i16(item_table)

    mesh = plsc.VectorSubcoreMesh(core_axis_name="c", subcore_axis_name="s")
    out_sds = jax.ShapeDtypeStruct((B, W), jnp.int16)

    @functools.partial(
        pl.kernel,
        mesh=mesh,
        compiler_params=pltpu.CompilerParams(use_tc_tiling_on_sc=False),
        out_type=(out_sds, out_sds, out_sds),
        scratch_types=[
            pltpu.VMEM((b_per_w,), jnp.int32),
            pltpu.VMEM((b_per_w,), jnp.int32),
            pltpu.VMEM((b_per_w,), jnp.int32),
            pltpu.VMEM((b_per_w, W), jnp.int16),
            pltpu.VMEM((b_per_w, W), jnp.int16),
            pltpu.VMEM((b_per_w, W), jnp.int16),
            pltpu.SemaphoreType.DMA,
            pltpu.SemaphoreType.DMA,
            pltpu.SemaphoreType.DMA,
        ],
    )
    def gather_kernel(ut_hbm, ui_hbm, pi_hbm, ni_hbm, it_hbm,
                      ou_hbm, op_hbm, on_hbm,
                      ui_v, pi_v, ni_v, ur_v, pr_v, nr_v,
                      sem_u, sem_p, sem_n):
        wid = lax.axis_index("s") * _NC + lax.axis_index("c")
        base = wid * b_per_w

        triples = (
            (ut_hbm, ui_hbm, ou_hbm, ui_v, ur_v, sem_u),
            (it_hbm, pi_hbm, op_hbm, pi_v, pr_v, sem_p),
            (it_hbm, ni_hbm, on_hbm, ni_v, nr_v, sem_n),
        )
        copies = []
        for tbl, idx_hbm, _out, idx_v, rows_v, sem in triples:
            pltpu.sync_copy(idx_hbm.at[pl.ds(base, b_per_w)], idx_v)
            copies.append(pltpu.async_copy(tbl.at[idx_v], rows_v, sem))
        for (_tbl, _idx, out_hbm, _iv, rows_v, _sem), cp in zip(triples, copies):
            cp.wait()
            pltpu.sync_copy(rows_v, out_hbm.at[pl.ds(base, b_per_w)])

    ou, op, on = gather_kernel(ut, u_idx, p_idx, n_idx, it)

    def _back(o):
        return jax.lax.bitcast_convert_type(o.reshape(B, D, 2), jnp.float32)

    return (_back(ou), _back(op), _back(on))


# speedup vs baseline: 4.3582x; 4.3582x over previous
import functools

import jax
import jax.numpy as jnp
from jax import lax
from jax.experimental import pallas as pl
from jax.experimental.pallas import tpu as pltpu
from jax.experimental.pallas import tpu_sc as plsc

_NC = 2
_NS = 16
_NW = _NC * _NS
_TC_ROWS = 512


def _sc_pair_gather(tp_u, tp_i, u_pidx, p_pidx, n_pidx, B):
    b_per_w = B // _NW
    mesh = plsc.VectorSubcoreMesh(core_axis_name="c", subcore_axis_name="s")
    out_sds = jax.ShapeDtypeStruct((B, 128), jnp.float32)

    @functools.partial(
        pl.kernel,
        mesh=mesh,
        out_type=(out_sds, out_sds, out_sds),
        scratch_types=[
            pltpu.VMEM((b_per_w,), jnp.int32),
            pltpu.VMEM((b_per_w, 128), jnp.float32),
            pltpu.SemaphoreType.DMA,
        ],
    )
    def gather_kernel(tu_hbm, ui_hbm, pi_hbm, ni_hbm, ti_hbm,
                      ou_hbm, op_hbm, on_hbm,
                      idx_v, dst_v, sem):
        wid = lax.axis_index("s") * _NC + lax.axis_index("c")
        base = wid * b_per_w

        for tbl, idx_hbm, out_hbm in (
            (tu_hbm, ui_hbm, ou_hbm),
            (ti_hbm, pi_hbm, op_hbm),
            (ti_hbm, ni_hbm, on_hbm),
        ):
            pltpu.sync_copy(idx_hbm.at[pl.ds(base, b_per_w)], idx_v)
            pltpu.async_copy(tbl.at[idx_v], dst_v, sem).wait()
            pltpu.sync_copy(dst_v, out_hbm.at[pl.ds(base, b_per_w)])

    return gather_kernel(tp_u, u_pidx, p_pidx, n_pidx, tp_i)


def _tc_half_select(pairs, parity, B, D):
    n_blk = B // _TC_ROWS
    par3 = parity.reshape(n_blk, 1, _TC_ROWS)

    def body(pair_ref, par_ref, out_ref):
        x = pair_ref[...]
        p = par_ref[0, 0, :].reshape(_TC_ROWS, 1)
        out_ref[...] = jnp.where(p == 1, x[:, D:2 * D], x[:, :D])

    return pl.pallas_call(
        body,
        grid=(n_blk,),
        in_specs=[
            pl.BlockSpec((_TC_ROWS, 2 * D), lambda i: (i, 0)),
            pl.BlockSpec((1, 1, _TC_ROWS), lambda i: (i, 0, 0)),
        ],
        out_specs=pl.BlockSpec((_TC_ROWS, D), lambda i: (i, 0)),
        out_shape=jax.ShapeDtypeStruct((B, D), jnp.float32),
    )(pairs, par3)


def kernel(users, pos_items, neg_items, user_table, item_table):
    B = users.shape[0]
    N, D = user_table.shape

    u_idx = users.astype(jnp.int32)
    p_idx = pos_items.astype(jnp.int32)
    n_idx = neg_items.astype(jnp.int32)

    tp_u = user_table.reshape(N // 2, 2 * D)
    tp_i = item_table.reshape(N // 2, 2 * D)

    pu, pp, pn = _sc_pair_gather(
        tp_u, tp_i, u_idx >> 1, p_idx >> 1, n_idx >> 1, B
    )
    return (
        _tc_half_select(pu, u_idx & 1, B, D),
        _tc_half_select(pp, p_idx & 1, B, D),
        _tc_half_select(pn, n_idx & 1, B, D),
    )
